# Initial kernel scaffold; baseline (speedup 1.0000x reference)
import jax
import jax.numpy as jnp
from jax.experimental import pallas as pl


def kernel(x, edge_index, W1, b1, W2, b2, p1, p2):
    raise NotImplementedError("write your pallas kernel here")


# speedup vs baseline: 3.5118x; 3.5118x over previous
import functools

import jax
import jax.numpy as jnp
from jax import lax
from jax.experimental import pallas as pl
from jax.experimental.pallas import tpu as pltpu
from jax.experimental.pallas import tpu_sc as plsc

F32 = jnp.float32
EPS = 1e-15
MAXNORM = 1.0 - 1e-5
NEG_BIG = -3.0e38

D = 64
WEXT = 128
N_NODES = 50000
N_EDGES = 800000
BLK = 1000
GRID = N_NODES // BLK

QN = N_NODES // 4
PADR = 8
TPT = 784
RPS = TPT * 16
CHR = TPT // 14
NZCH = TPT // CHR
EC = 128
ECHUNKS = N_EDGES // EC


def _nrm(x):
    return jnp.maximum(jnp.sqrt(jnp.sum(x * x, axis=-1, keepdims=True)), EPS)


def _artanh(x):
    xc = jnp.clip(x, -1.0 + 1e-7, 1.0 - 1e-7)
    return 0.5 * (jnp.log1p(xc) - jnp.log1p(-xc))


def _proj(x):
    n = _nrm(x)
    return jnp.where(n > MAXNORM, x / n * MAXNORM, x)


def _expmap0(u):
    un = _nrm(u)
    return _proj(jnp.tanh(un) * u / un)


def _logmap0(p):
    pn = _nrm(p)
    return _artanh(pn) * p / pn


def _mobius_add(x, y):
    x2 = jnp.sum(x * x, axis=-1, keepdims=True)
    y2 = jnp.sum(y * y, axis=-1, keepdims=True)
    xy = jnp.sum(x * y, axis=-1, keepdims=True)
    num = (1.0 + 2.0 * xy + y2) * x + (1.0 - x2) * y
    den = 1.0 + 2.0 * xy + x2 * y2
    return num / jnp.maximum(den, EPS)


def _mobius_matvec(x, w):
    xn = _nrm(x)
    mx = lax.dot_general(x, w, (((1,), (1,)), ((), ())),
                         preferred_element_type=F32)
    mxn = _nrm(mx)
    return jnp.tanh(mxn / xn * _artanh(xn)) * mx / mxn


def _hyplinear_tangent(x, w, b):
    mv = _proj(_mobius_matvec(x, w))
    hb = _proj(_expmap0(b))
    h = _proj(_mobius_add(mv, hb))
    return _logmap0(h)


def _tca_body(x_ref, w_ref, b_ref, o_ref):
    x = x_ref[...]
    xh = _proj(_expmap0(x))
    xt = _hyplinear_tangent(xh, w_ref[...], b_ref[...])
    o_ref[...] = jnp.concatenate(
        [xt, jnp.ones((BLK, 1), F32), jnp.zeros((BLK, WEXT - D - 1), F32)],
        axis=1)


def _tcb_body(a_ref, p_ref, xtp_ref, s_ref):
    a = a_ref[...]
    agg = a[:, :D] / jnp.maximum(a[:, D:D + 1], 1.0)
    h1 = _proj(_expmap0(agg))
    xtb = jnp.maximum(_logmap0(h1), 0.0)
    h1f = _proj(_expmap0(xtb))
    xtp = _logmap0(h1f)
    p = p_ref[...]
    phat = p / jnp.maximum(jnp.sqrt(jnp.sum(p * p)), EPS)
    s = jnp.sum(xtp * phat, axis=-1, keepdims=True)
    xtp_ref[...] = xtp
    s_ref[...] = jnp.broadcast_to(s, (BLK, 8))


def _tcc_body(xtp_ref, s_ref, sel_ref, w_ref, b_ref, o_ref):
    xtp = xtp_ref[...]
    s = s_ref[:, :1]
    sel = sel_ref[:, :1]
    row = xtp * jnp.tanh(s) * sel
    x1 = _proj(_expmap0(row))
    xt2 = _hyplinear_tangent(x1, w_ref[...], b_ref[...])
    o_ref[...] = jnp.concatenate(
        [xt2 * sel, jnp.broadcast_to(sel, (BLK, 1)),
         jnp.zeros((BLK, WEXT - D - 1), F32)],
        axis=1)


def _tcd_body(a_ref, sel_ref, p_ref, xt_ref, s_ref):
    a = a_ref[...]
    agg = a[:, :D] / jnp.maximum(a[:, D:D + 1], 1.0)
    h2 = _proj(_expmap0(agg))
    xtb = jnp.maximum(_logmap0(h2), 0.0)
    h2f = _proj(_expmap0(xtb))
    xt2t = _logmap0(h2f)
    p = p_ref[...]
    phat = p / jnp.maximum(jnp.sqrt(jnp.sum(p * p)), EPS)
    s2 = jnp.sum(xt2t * phat, axis=-1, keepdims=True)
    s2m = jnp.where(sel_ref[:, :1] > 0.0, s2, NEG_BIG)
    xt_ref[...] = xt2t
    s_ref[...] = jnp.broadcast_to(s2m, (BLK, 8))


def _tce_body(xt_ref, s_ref, sel_ref, o_ref):
    i = pl.program_id(0)
    r = xt_ref[...] * jnp.tanh(s_ref[:, :1])
    sel = sel_ref[:, :1] > 0.0
    psum = jnp.sum(jnp.where(sel, r, 0.0), axis=0, keepdims=True)
    pmax = jnp.max(jnp.where(sel, r, NEG_BIG), axis=0, keepdims=True)

    @pl.when(i == 0)
    def _():
        o_ref[...] = jnp.zeros((8, D), F32)
        o_ref[0:1, :] = psum
        o_ref[1:2, :] = pmax

    @pl.when(i > 0)
    def _():
        o_ref[0:1, :] = o_ref[0:1, :] + psum
        o_ref[1:2, :] = jnp.maximum(o_ref[1:2, :], pmax)


def _row_spec(w):
    return pl.BlockSpec((BLK, w), lambda i: (i, 0))


def _fix_spec(r, w):
    return pl.BlockSpec((r, w), lambda i: (0, 0))


_tca = pl.pallas_call(
    _tca_body, grid=(GRID,),
    in_specs=[_row_spec(D), _fix_spec(D, D), _fix_spec(1, D)],
    out_specs=_row_spec(WEXT),
    out_shape=jax.ShapeDtypeStruct((N_NODES, WEXT), F32))

_tcb = pl.pallas_call(
    _tcb_body, grid=(GRID,),
    in_specs=[_row_spec(WEXT), _fix_spec(1, D)],
    out_specs=[_row_spec(D), _row_spec(8)],
    out_shape=[jax.ShapeDtypeStruct((N_NODES, D), F32),
               jax.ShapeDtypeStruct((N_NODES, 8), F32)])

_tcc = pl.pallas_call(
    _tcc_body, grid=(GRID,),
    in_specs=[_row_spec(D), _row_spec(8), _row_spec(8),
              _fix_spec(D, D), _fix_spec(1, D)],
    out_specs=_row_spec(WEXT),
    out_shape=jax.ShapeDtypeStruct((N_NODES, WEXT), F32))

_tcd = pl.pallas_call(
    _tcd_body, grid=(GRID,),
    in_specs=[_row_spec(WEXT), _row_spec(8), _fix_spec(1, D)],
    out_specs=[_row_spec(D), _row_spec(8)],
    out_shape=[jax.ShapeDtypeStruct((N_NODES, D), F32),
               jax.ShapeDtypeStruct((N_NODES, 8), F32)])

_tce = pl.pallas_call(
    _tce_body, grid=(GRID,),
    in_specs=[_row_spec(D), _row_spec(8), _row_spec(8)],
    out_specs=pl.BlockSpec((8, D), lambda i: (0, 0)),
    out_shape=jax.ShapeDtypeStruct((8, D), F32))


def _sc_agg(table, src, dst):
    mesh = plsc.VectorSubcoreMesh(core_axis_name="c", subcore_axis_name="s")

    @functools.partial(
        pl.kernel,
        mesh=mesh,
        out_type=jax.ShapeDtypeStruct((4, RPS, WEXT), F32),
        scratch_types=[
            pltpu.VMEM_SHARED((RPS, WEXT), F32),
            pltpu.VMEM((CHR, WEXT), F32),
            pltpu.VMEM((EC, WEXT), F32),
            pltpu.VMEM((EC,), jnp.int32),
            pltpu.VMEM((EC,), jnp.int32),
            pltpu.VMEM((EC,), jnp.int32),
            pltpu.SemaphoreType.DMA,
        ],
    )
    def k(table_h, src_h, dst_h, out_h, acc, zbuf, rows, sidx, didx, lidx,
          sem):
        core = lax.axis_index("c")
        t = lax.axis_index("s")

        def zrow(r, _):
            for j in range(WEXT // 16):
                zbuf[r, pl.ds(j * 16, 16)] = jnp.zeros((16,), F32)
            return _

        spread = lax.iota(jnp.int32, 16) & (PADR - 1)
        base_chunks = ECHUNKS // 16
        nchunk = base_chunks + jnp.where(t < ECHUNKS - base_chunks * 16,
                                         1, 0)

        for p in range(2):
            q = 2 * core + p
            base = q * QN
            lax.fori_loop(0, CHR, zrow, None)
            for z in range(NZCH):
                pltpu.sync_copy(zbuf, acc.at[pl.ds(t * TPT + z * CHR, CHR)])
            plsc.subcore_barrier()

            def body(g, _):
                e0 = (g * 16 + t) * EC
                pltpu.sync_copy(src_h.at[pl.ds(e0, EC)], sidx)
                pltpu.sync_copy(dst_h.at[pl.ds(e0, EC)], didx)
                pltpu.async_copy(table_h.at[sidx], rows, sem).wait()
                for i in range(EC // 16):
                    dd = didx[pl.ds(i * 16, 16)]
                    rel = dd - base
                    ok = (rel >= 0) & (rel < QN)
                    lidx[pl.ds(i * 16, 16)] = jnp.where(ok, rel,
                                                        QN + spread)
                pltpu.sync_copy(rows, acc.at[lidx], add=True)
                return _

            lax.fori_loop(0, nchunk, body, None)
            plsc.subcore_barrier()

            for z in range(NZCH):
                pltpu.sync_copy(acc.at[pl.ds(t * TPT + z * CHR, CHR)], zbuf)
                pltpu.sync_copy(zbuf, out_h.at[q, pl.ds(t * TPT + z * CHR,
                                                        CHR)])
            plsc.subcore_barrier()

    out = k(table, src, dst)
    return jnp.concatenate([out[q, :QN] for q in range(4)], axis=0)


def kernel(x, edge_index, W1, b1, W2, b2, p1, p2):
    n, d = x.shape
    k1 = n // 2
    k2 = k1 // 2
    src, dst = edge_index[0], edge_index[1]
    b1r = b1.reshape(1, d)
    b2r = b2.reshape(1, d)
    p1r = p1.reshape(1, d)
    p2r = p2.reshape(1, d)

    ext1 = _tca(x, W1, b1r)
    agg1 = _sc_agg(ext1, src, dst)
    xtp, s1b = _tcb(agg1, p1r)

    s1 = s1b[:, 0]
    _, idx1 = lax.top_k(s1, k1)
    sel1 = jnp.zeros((n,), F32).at[idx1].set(1.0)
    sel1b = jnp.broadcast_to(sel1[:, None], (n, 8))

    ext2 = _tcc(xtp, s1b, sel1b, W2, b2r)
    agg2 = _sc_agg(ext2, src, dst)
    xt2t, s2b = _tcd(agg2, sel1b, p2r)

    s2 = s2b[:, 0]
    _, idx2 = lax.top_k(s2, k2)
    sel2 = jnp.zeros((n,), F32).at[idx2].set(1.0)
    sel2b = jnp.broadcast_to(sel2[:, None], (n, 8))

    red = _tce(xt2t, s2b, sel2b)
    return jnp.concatenate([red[0] / k2, red[1]], axis=0)


# speedup vs baseline: 4.4116x; 1.2562x over previous
import functools

import jax
import jax.numpy as jnp
from jax import lax
from jax.experimental import pallas as pl
from jax.experimental.pallas import tpu as pltpu
from jax.experimental.pallas import tpu_sc as plsc

F32 = jnp.float32
EPS = 1e-15
MAXNORM = 1.0 - 1e-5
NEG_BIG = -3.0e38

D = 64
WEXT = 128
N_NODES = 50000
N_EDGES = 800000
BLK = 1000
GRID = N_NODES // BLK

QN = N_NODES // 4
PADR = 8
TPT = 784
RPS = TPT * 16
CHR = TPT // 14
NZCH = TPT // CHR
EC = 80
EPT = N_EDGES // 16
NCHUNK = EPT // EC
NPAIR = NCHUNK // 2


def _nrm(x):
    return jnp.maximum(jnp.sqrt(jnp.sum(x * x, axis=-1, keepdims=True)), EPS)


def _artanh(x):
    xc = jnp.clip(x, -1.0 + 1e-7, 1.0 - 1e-7)
    return 0.5 * (jnp.log1p(xc) - jnp.log1p(-xc))


def _proj(x):
    n = _nrm(x)
    return jnp.where(n > MAXNORM, x / n * MAXNORM, x)


def _expmap0(u):
    un = _nrm(u)
    return _proj(jnp.tanh(un) * u / un)


def _logmap0(p):
    pn = _nrm(p)
    return _artanh(pn) * p / pn


def _mobius_add(x, y):
    x2 = jnp.sum(x * x, axis=-1, keepdims=True)
    y2 = jnp.sum(y * y, axis=-1, keepdims=True)
    xy = jnp.sum(x * y, axis=-1, keepdims=True)
    num = (1.0 + 2.0 * xy + y2) * x + (1.0 - x2) * y
    den = 1.0 + 2.0 * xy + x2 * y2
    return num / jnp.maximum(den, EPS)


def _mobius_matvec(x, w):
    xn = _nrm(x)
    mx = lax.dot_general(x, w, (((1,), (1,)), ((), ())),
                         preferred_element_type=F32)
    mxn = _nrm(mx)
    return jnp.tanh(mxn / xn * _artanh(xn)) * mx / mxn


def _hyplinear_tangent(x, w, b):
    mv = _proj(_mobius_matvec(x, w))
    hb = _proj(_expmap0(b))
    h = _proj(_mobius_add(mv, hb))
    return _logmap0(h)


def _tca_body(x_ref, w_ref, b_ref, o_ref):
    x = x_ref[...]
    xh = _proj(_expmap0(x))
    xt = _hyplinear_tangent(xh, w_ref[...], b_ref[...])
    o_ref[...] = jnp.concatenate(
        [xt, jnp.ones((BLK, 1), F32), jnp.zeros((BLK, WEXT - D - 1), F32)],
        axis=1)


def _tcb_body(a_ref, p_ref, xtp_ref, s_ref):
    a = a_ref[...]
    agg = a[:, :D] / jnp.maximum(a[:, D:D + 1], 1.0)
    h1 = _proj(_expmap0(agg))
    xtb = jnp.maximum(_logmap0(h1), 0.0)
    h1f = _proj(_expmap0(xtb))
    xtp = _logmap0(h1f)
    p = p_ref[...]
    phat = p / jnp.maximum(jnp.sqrt(jnp.sum(p * p)), EPS)
    s = jnp.sum(xtp * phat, axis=-1, keepdims=True)
    xtp_ref[...] = xtp
    s_ref[...] = jnp.broadcast_to(s, (BLK, 8))


def _tcc_body(xtp_ref, s_ref, sel_ref, w_ref, b_ref, o_ref):
    xtp = xtp_ref[...]
    s = s_ref[:, :1]
    sel = sel_ref[:, :1]
    row = xtp * jnp.tanh(s) * sel
    x1 = _proj(_expmap0(row))
    xt2 = _hyplinear_tangent(x1, w_ref[...], b_ref[...])
    o_ref[...] = jnp.concatenate(
        [xt2 * sel, jnp.broadcast_to(sel, (BLK, 1)),
         jnp.zeros((BLK, WEXT - D - 1), F32)],
        axis=1)


def _tcd_body(a_ref, sel_ref, p_ref, xt_ref, s_ref):
    a = a_ref[...]
    agg = a[:, :D] / jnp.maximum(a[:, D:D + 1], 1.0)
    h2 = _proj(_expmap0(agg))
    xtb = jnp.maximum(_logmap0(h2), 0.0)
    h2f = _proj(_expmap0(xtb))
    xt2t = _logmap0(h2f)
    p = p_ref[...]
    phat = p / jnp.maximum(jnp.sqrt(jnp.sum(p * p)), EPS)
    s2 = jnp.sum(xt2t * phat, axis=-1, keepdims=True)
    s2m = jnp.where(sel_ref[:, :1] > 0.0, s2, NEG_BIG)
    xt_ref[...] = xt2t
    s_ref[...] = jnp.broadcast_to(s2m, (BLK, 8))


def _tce_body(xt_ref, s_ref, sel_ref, o_ref):
    i = pl.program_id(0)
    r = xt_ref[...] * jnp.tanh(s_ref[:, :1])
    sel = sel_ref[:, :1] > 0.0
    psum = jnp.sum(jnp.where(sel, r, 0.0), axis=0, keepdims=True)
    pmax = jnp.max(jnp.where(sel, r, NEG_BIG), axis=0, keepdims=True)

    @pl.when(i == 0)
    def _():
        o_ref[...] = jnp.zeros((8, D), F32)
        o_ref[0:1, :] = psum
        o_ref[1:2, :] = pmax

    @pl.when(i > 0)
    def _():
        o_ref[0:1, :] = o_ref[0:1, :] + psum
        o_ref[1:2, :] = jnp.maximum(o_ref[1:2, :], pmax)


def _row_spec(w):
    return pl.BlockSpec((BLK, w), lambda i: (i, 0))


def _fix_spec(r, w):
    return pl.BlockSpec((r, w), lambda i: (0, 0))


_tca = pl.pallas_call(
    _tca_body, grid=(GRID,),
    in_specs=[_row_spec(D), _fix_spec(D, D), _fix_spec(1, D)],
    out_specs=_row_spec(WEXT),
    out_shape=jax.ShapeDtypeStruct((N_NODES, WEXT), F32))

_tcb = pl.pallas_call(
    _tcb_body, grid=(GRID,),
    in_specs=[_row_spec(WEXT), _fix_spec(1, D)],
    out_specs=[_row_spec(D), _row_spec(8)],
    out_shape=[jax.ShapeDtypeStruct((N_NODES, D), F32),
               jax.ShapeDtypeStruct((N_NODES, 8), F32)])

_tcc = pl.pallas_call(
    _tcc_body, grid=(GRID,),
    in_specs=[_row_spec(D), _row_spec(8), _row_spec(8),
              _fix_spec(D, D), _fix_spec(1, D)],
    out_specs=_row_spec(WEXT),
    out_shape=jax.ShapeDtypeStruct((N_NODES, WEXT), F32))

_tcd = pl.pallas_call(
    _tcd_body, grid=(GRID,),
    in_specs=[_row_spec(WEXT), _row_spec(8), _fix_spec(1, D)],
    out_specs=[_row_spec(D), _row_spec(8)],
    out_shape=[jax.ShapeDtypeStruct((N_NODES, D), F32),
               jax.ShapeDtypeStruct((N_NODES, 8), F32)])

_tce = pl.pallas_call(
    _tce_body, grid=(GRID,),
    in_specs=[_row_spec(D), _row_spec(8), _row_spec(8)],
    out_specs=pl.BlockSpec((8, D), lambda i: (0, 0)),
    out_shape=jax.ShapeDtypeStruct((8, D), F32))


def _sc_agg(table, src, dst):
    mesh = plsc.VectorSubcoreMesh(core_axis_name="c", subcore_axis_name="s")

    @functools.partial(
        pl.kernel,
        mesh=mesh,
        out_type=jax.ShapeDtypeStruct((4, RPS, WEXT), F32),
        scratch_types=[
            pltpu.VMEM_SHARED((RPS, WEXT), F32),
            pltpu.VMEM((CHR, WEXT), F32),
            pltpu.VMEM((EC, WEXT), F32),
            pltpu.VMEM((EC, WEXT), F32),
            pltpu.VMEM((EC,), jnp.int32),
            pltpu.VMEM((EC,), jnp.int32),
            pltpu.VMEM((EC,), jnp.int32),
            pltpu.VMEM((EC,), jnp.int32),
            pltpu.VMEM((EC,), jnp.int32),
            pltpu.SemaphoreType.DMA,
            pltpu.SemaphoreType.DMA,
        ],
    )
    def k(table_h, src_h, dst_h, out_h, acc, zbuf, rows0, rows1, sidx0,
          sidx1, didx0, didx1, lidx, sem0, sem1):
        core = lax.axis_index("c")
        t = lax.axis_index("s")
        rows = [rows0, rows1]
        sidx = [sidx0, sidx1]
        didx = [didx0, didx1]
        sem = [sem0, sem1]

        def zrow(r, _):
            for j in range(WEXT // 16):
                zbuf[r, pl.ds(j * 16, 16)] = jnp.zeros((16,), F32)
            return _

        spread = lax.iota(jnp.int32, 16) & (PADR - 1)
        ebase = t * EPT

        for p in range(2):
            q = 2 * core + p
            base = q * QN
            lax.fori_loop(0, CHR, zrow, None)
            for z in range(NZCH):
                pltpu.sync_copy(zbuf, acc.at[pl.ds(t * TPT + z * CHR, CHR)])
            plsc.subcore_barrier()

            def fire(g, b):
                e0 = ebase + g * EC
                pltpu.sync_copy(src_h.at[pl.ds(e0, EC)], sidx[b])
                pltpu.sync_copy(dst_h.at[pl.ds(e0, EC)], didx[b])
                pltpu.make_async_copy(table_h.at[sidx[b]], rows[b],
                                      sem[b]).start()

            def drain(b):
                pltpu.make_async_copy(table_h.at[sidx[b]], rows[b],
                                      sem[b]).wait()
                for i in range(EC // 16):
                    dd = didx[b][pl.ds(i * 16, 16)]
                    rel = dd - base
                    ok = (rel >= 0) & (rel < QN)
                    lidx[pl.ds(i * 16, 16)] = jnp.where(ok, rel,
                                                        QN + spread)
                pltpu.sync_copy(rows[b], acc.at[lidx], add=True)

            fire(0, 0)

            def body(i, _):
                g = i * 2
                fire(g + 1, 1)
                drain(0)
                fire(g + 2, 0)
                drain(1)
                return _

            lax.fori_loop(0, NPAIR, body, None)
            drain(0)
            plsc.subcore_barrier()

            for z in range(NZCH):
                pltpu.sync_copy(acc.at[pl.ds(t * TPT + z * CHR, CHR)], zbuf)
                pltpu.sync_copy(zbuf, out_h.at[q, pl.ds(t * TPT + z * CHR,
                                                        CHR)])
            plsc.subcore_barrier()

    out = k(table, src, dst)
    return jnp.concatenate([out[q, :QN] for q in range(4)], axis=0)


def kernel(x, edge_index, W1, b1, W2, b2, p1, p2):
    n, d = x.shape
    k1 = n // 2
    k2 = k1 // 2
    src, dst = edge_index[0], edge_index[1]
    b1r = b1.reshape(1, d)
    b2r = b2.reshape(1, d)
    p1r = p1.reshape(1, d)
    p2r = p2.reshape(1, d)

    ext1 = _tca(x, W1, b1r)
    agg1 = _sc_agg(ext1, src, dst)
    xtp, s1b = _tcb(agg1, p1r)

    s1 = s1b[:, 0]
    _, idx1 = lax.top_k(s1, k1)
    sel1 = jnp.zeros((n,), F32).at[idx1].set(1.0)
    sel1b = jnp.broadcast_to(sel1[:, None], (n, 8))

    ext2 = _tcc(xtp, s1b, sel1b, W2, b2r)
    agg2 = _sc_agg(ext2, src, dst)
    xt2t, s2b = _tcd(agg2, sel1b, p2r)

    s2 = s2b[:, 0]
    _, idx2 = lax.top_k(s2, k2)
    sel2 = jnp.zeros((n,), F32).at[idx2].set(1.0)
    sel2b = jnp.broadcast_to(sel2[:, None], (n, 8))

    red = _tce(xt2t, s2b, sel2b)
    return jnp.concatenate([red[0] / k2, red[1]], axis=0)
